# baseline (speedup 1.0000x reference)
import functools

import jax
import jax.numpy as jnp
from jax import lax
from jax.experimental import pallas as pl
from jax.experimental.pallas import tpu as pltpu
from jax.experimental.pallas import tpu_sc as plsc

_N = 100000
_NE = 1600000
_EMB = 32
_HALF = 16
_NLAYERS = 3
_TILES = 16

_KE = 160
_ECHUNKS = _NE // _TILES // _KE
_KC = 1000
_CCHUNKS = _NE // 2 // _TILES // _KC
_ZCH = _N // _KE
_ZITERS = -(-_ZCH // _TILES)

_NB = 4000
_EBLK = 2000

_sc_mesh = plsc.VectorSubcoreMesh(core_axis_name="c", subcore_axis_name="s")
_sc_params = pltpu.CompilerParams(use_tc_tiling_on_sc=False)


@functools.partial(
    pl.kernel,
    out_type=jax.ShapeDtypeStruct((2 * _N, _HALF), jnp.float32),
    mesh=_sc_mesh,
    compiler_params=_sc_params,
    scratch_types=[
        pltpu.VMEM((2, _KE), jnp.int32),
        pltpu.VMEM((2, _KE), jnp.int32),
        pltpu.VMEM((2, _KE), jnp.int32),
        pltpu.VMEM((2, _KE), jnp.int32),
        pltpu.VMEM((2, _KE, _HALF), jnp.float32),
        pltpu.VMEM((2, _KE, _HALF), jnp.float32),
        pltpu.VMEM((2, _KE), jnp.float32),
        pltpu.VMEM((2, _KE), jnp.float32),
        pltpu.VMEM((2, _KE, _HALF), jnp.float32),
        pltpu.VMEM((2, _HALF), jnp.float32),
        pltpu.SemaphoreType.DMA,
        pltpu.SemaphoreType.DMA,
        pltpu.SemaphoreType.DMA,
        pltpu.SemaphoreType.DMA,
        pltpu.SemaphoreType.DMA,
        pltpu.VMEM_SHARED((_N, _HALF), jnp.float32),
    ],
)
def _sc_edge_pass(a_tab, b_tab, ef0_hbm, ef1_hbm, wv_hbm, rows_hbm, cols_hbm,
                  out_hbm,
                  idx_r, idx_s, idx_ro, idx_co, abuf, bbuf, e0buf, e1buf, mbuf,
                  wbuf, sem_i, sem_a, sem_b, sem_e, sem_s, acc):
    c = lax.axis_index("c")
    t = lax.axis_index("s")
    pltpu.sync_copy(wv_hbm.at[c], wbuf)
    w0 = wbuf[0]
    w1 = wbuf[1]

    def _zrow(j, carry):
        mbuf[0, j] = jnp.zeros((_HALF,), jnp.float32)
        return carry
    lax.fori_loop(0, _KE, _zrow, 0)

    def _zchunk(m_i, carry):
        m = m_i * _TILES + t

        @pl.when(m < _ZCH)
        def _():
            pltpu.sync_copy(mbuf.at[0], acc.at[pl.ds(m * _KE, _KE)])
        return carry
    lax.fori_loop(0, _ZITERS, _zchunk, 0)
    plsc.subcore_barrier()

    ebase = t * (_NE // _TILES)
    coff = c * _N

    def _prefetch(k, p):
        off = ebase + k * _KE
        cp_r = pltpu.async_copy(rows_hbm.at[pl.ds(off, _KE)], idx_r.at[p],
                                sem_i)
        cp_c = pltpu.async_copy(cols_hbm.at[pl.ds(off, _KE)], idx_co.at[p],
                                sem_i)
        cp_r.wait()
        cp_c.wait()
        for j in range(_KE // 16):
            s = pl.ds(j * 16, 16)
            idx_ro[p, s] = idx_r[p, s] + coff
            idx_co[p, s] = idx_co[p, s] + coff
        pltpu.async_copy(a_tab.at[idx_ro.at[p]], abuf.at[p], sem_a)
        pltpu.async_copy(b_tab.at[idx_co.at[p]], bbuf.at[p], sem_b)
        pltpu.async_copy(ef0_hbm.at[pl.ds(off, _KE)], e0buf.at[p], sem_e)
        pltpu.async_copy(ef1_hbm.at[pl.ds(off, _KE)], e1buf.at[p], sem_e)

    _prefetch(0, 0)
    _prefetch(1, 1)

    def _chunk(k, carry):
        p = lax.rem(k, 2)
        pltpu.make_async_copy(a_tab.at[idx_ro.at[p]], abuf.at[p], sem_a).wait()
        pltpu.make_async_copy(b_tab.at[idx_co.at[p]], bbuf.at[p], sem_b).wait()
        pltpu.make_async_copy(ef0_hbm.at[pl.ds(0, _KE)], e0buf.at[p],
                              sem_e).wait()
        pltpu.make_async_copy(ef1_hbm.at[pl.ds(0, _KE)], e1buf.at[p],
                              sem_e).wait()

        @pl.when(k > 0)
        def _():
            pltpu.make_async_copy(mbuf.at[1 - p],
                                  acc.at[idx_s.at[1 - p]], sem_s).wait()

        for j in range(_KE // 16):
            s = pl.ds(j * 16, 16)
            idx_s[p, s] = idx_r[p, s]

        def _comp(g, cc):
            es0 = e0buf[p, pl.ds(g * 16, 16)]
            es1 = e1buf[p, pl.ds(g * 16, 16)]
            for u in range(16):
                j = g * 16 + u
                ev = es0[u] * w0 + es1[u] * w1
                mbuf[p, j] = jnp.maximum(abuf[p, j] + bbuf[p, j] + ev, 0.0)
            return cc
        lax.fori_loop(0, _KE // 16, _comp, 0)

        pltpu.async_copy(mbuf.at[p], acc.at[idx_s.at[p]], sem_s, add=True)

        @pl.when(k < _ECHUNKS - 2)
        def _():
            _prefetch(k + 2, p)
        return carry
    lax.fori_loop(0, _ECHUNKS, _chunk, 0)
    pf = (_ECHUNKS - 1) % 2
    pltpu.make_async_copy(mbuf.at[pf], acc.at[idx_s.at[pf]], sem_s).wait()
    plsc.subcore_barrier()

    def _wchunk(m_i, carry):
        m = m_i * _TILES + t

        @pl.when(m < _ZCH)
        def _():
            pltpu.sync_copy(acc.at[pl.ds(m * _KE, _KE)],
                            out_hbm.at[pl.ds(c * _N + m * _KE, _KE)])
        return carry
    lax.fori_loop(0, _ZITERS, _wchunk, 0)


@functools.partial(
    pl.kernel,
    out_type=jax.ShapeDtypeStruct((2 * _N, _HALF), jnp.float32),
    mesh=_sc_mesh,
    compiler_params=_sc_params,
    scratch_types=[
        pltpu.VMEM((_KC,), jnp.int32),
        pltpu.VMEM((_KC, _HALF), jnp.float32),
        pltpu.SemaphoreType.DMA,
        pltpu.VMEM_SHARED((_N, _HALF), jnp.float32),
    ],
)
def _sc_count_pass(rows_hbm, out_hbm, idx_r, vbuf, sem, acc):
    c = lax.axis_index("c")
    t = lax.axis_index("s")

    def _zrow(j, carry):
        vbuf[j] = jnp.zeros((_HALF,), jnp.float32)
        return carry
    lax.fori_loop(0, _KC, _zrow, 0)

    def _zchunk(m_i, carry):
        m = m_i * _TILES + t

        @pl.when(m < _N // _KC)
        def _():
            pltpu.sync_copy(vbuf, acc.at[pl.ds(m * _KC, _KC)])
        return carry
    lax.fori_loop(0, -(-(_N // _KC) // _TILES), _zchunk, 0)
    plsc.subcore_barrier()

    def _orow(j, carry):
        vbuf[j] = jnp.full((_HALF,), 1.0, jnp.float32)
        return carry
    lax.fori_loop(0, _KC, _orow, 0)

    ebase = c * (_NE // 2) + t * (_NE // 2 // _TILES)

    def _chunk(k, carry):
        off = ebase + k * _KC
        pltpu.sync_copy(rows_hbm.at[pl.ds(off, _KC)], idx_r)
        pltpu.sync_copy(vbuf, acc.at[idx_r], add=True)
        return carry
    lax.fori_loop(0, _CCHUNKS, _chunk, 0)
    plsc.subcore_barrier()

    def _wchunk(m_i, carry):
        m = m_i * _TILES + t

        @pl.when(m < _N // _KC)
        def _():
            pltpu.sync_copy(acc.at[pl.ds(m * _KC, _KC)],
                            out_hbm.at[pl.ds(c * _N + m * _KC, _KC)])
        return carry
    lax.fori_loop(0, -(-(_N // _KC) // _TILES), _wchunk, 0)


def _enc_body(nf_ref, w1, b1, w2, b2, wr, wc, cb1, h_ref, a_ref, b_ref):
    x = nf_ref[...]
    tmp = jnp.maximum(
        jnp.dot(x, w1[...], preferred_element_type=jnp.float32) + b1[...], 0.0)
    h = jnp.dot(tmp, w2[...], preferred_element_type=jnp.float32) + b2[...]
    h_ref[...] = h
    p = jnp.dot(h, wr[...], preferred_element_type=jnp.float32) + cb1[...]
    q = jnp.dot(h, wc[...], preferred_element_type=jnp.float32)
    a_ref[0] = p[:, :_HALF]
    a_ref[1] = p[:, _HALF:]
    b_ref[0] = q[:, :_HALF]
    b_ref[1] = q[:, _HALF:]


def _enc_call(nf, w1, b1, w2, b2, wr, wc, cb1):
    hid = w1.shape[1]
    return pl.pallas_call(
        _enc_body,
        grid=(_N // _NB,),
        in_specs=[
            pl.BlockSpec((_NB, nf.shape[1]), lambda i: (i, 0)),
            pl.BlockSpec(w1.shape, lambda i: (0, 0)),
            pl.BlockSpec((1, hid), lambda i: (0, 0)),
            pl.BlockSpec(w2.shape, lambda i: (0, 0)),
            pl.BlockSpec((1, _EMB), lambda i: (0, 0)),
            pl.BlockSpec((_EMB, _EMB), lambda i: (0, 0)),
            pl.BlockSpec((_EMB, _EMB), lambda i: (0, 0)),
            pl.BlockSpec((1, _EMB), lambda i: (0, 0)),
        ],
        out_specs=[
            pl.BlockSpec((_NB, _EMB), lambda i: (i, 0)),
            pl.BlockSpec((2, _NB, _HALF), lambda i: (0, i, 0)),
            pl.BlockSpec((2, _NB, _HALF), lambda i: (0, i, 0)),
        ],
        out_shape=[
            jax.ShapeDtypeStruct((_N, _EMB), jnp.float32),
            jax.ShapeDtypeStruct((2, _N, _HALF), jnp.float32),
            jax.ShapeDtypeStruct((2, _N, _HALF), jnp.float32),
        ],
    )(nf, w1, b1, w2, b2, wr, wc, cb1)


def _node_update(agg_ref, cnt_ref, h_ref, w2, b2, skw, skb, g, beta):
    agg = jnp.concatenate([agg_ref[0], agg_ref[1]], axis=-1)
    cv = cnt_ref[0, :, 0:1] + cnt_ref[1, :, 0:1]
    h = h_ref[...]
    z = (jnp.dot(agg, w2[...], preferred_element_type=jnp.float32)
         + cv * b2[...]
         + jnp.dot(h, skw[...], preferred_element_type=jnp.float32)
         + skb[...])
    mu = jnp.mean(z, axis=-1, keepdims=True)
    var = jnp.mean(jnp.square(z - mu), axis=-1, keepdims=True)
    zn = (z - mu) * lax.rsqrt(var + 1e-5) * g[...] + beta[...]
    return jnp.maximum(zn, 0.0)


def _upd_body(agg_ref, cnt_ref, h_ref, w2, b2, skw, skb, g, beta, wr, wc, cb1,
              hn_ref, a_ref, b_ref):
    hn = _node_update(agg_ref, cnt_ref, h_ref, w2, b2, skw, skb, g, beta)
    hn_ref[...] = hn
    p = jnp.dot(hn, wr[...], preferred_element_type=jnp.float32) + cb1[...]
    q = jnp.dot(hn, wc[...], preferred_element_type=jnp.float32)
    a_ref[0] = p[:, :_HALF]
    a_ref[1] = p[:, _HALF:]
    b_ref[0] = q[:, :_HALF]
    b_ref[1] = q[:, _HALF:]


def _upd_call(agg, cnt, h, w2, b2, skw, skb, g, beta, wr, wc, cb1):
    wspec = pl.BlockSpec((_EMB, _EMB), lambda i: (0, 0))
    vspec = pl.BlockSpec((1, _EMB), lambda i: (0, 0))
    return pl.pallas_call(
        _upd_body,
        grid=(_N // _NB,),
        in_specs=[
            pl.BlockSpec((2, _NB, _HALF), lambda i: (0, i, 0)),
            pl.BlockSpec((2, _NB, _HALF), lambda i: (0, i, 0)),
            pl.BlockSpec((_NB, _EMB), lambda i: (i, 0)),
            wspec, vspec, wspec, vspec, vspec, vspec, wspec, wspec, vspec,
        ],
        out_specs=[
            pl.BlockSpec((_NB, _EMB), lambda i: (i, 0)),
            pl.BlockSpec((2, _NB, _HALF), lambda i: (0, i, 0)),
            pl.BlockSpec((2, _NB, _HALF), lambda i: (0, i, 0)),
        ],
        out_shape=[
            jax.ShapeDtypeStruct((_N, _EMB), jnp.float32),
            jax.ShapeDtypeStruct((2, _N, _HALF), jnp.float32),
            jax.ShapeDtypeStruct((2, _N, _HALF), jnp.float32),
        ],
    )(agg, cnt, h, w2, b2, skw, skb, g, beta, wr, wc, cb1)


def _fin_body(agg_ref, cnt_ref, h_ref, w2, b2, skw, skb, g, beta,
              hw1, hb1, hw2, hb2, out_ref):
    hn = _node_update(agg_ref, cnt_ref, h_ref, w2, b2, skw, skb, g, beta)
    tt = jnp.maximum(
        jnp.dot(hn, hw1[...], preferred_element_type=jnp.float32) + hb1[...],
        0.0)
    out_ref[...] = jnp.dot(tt, hw2[...], preferred_element_type=jnp.float32) + hb2[...]


def _fin_call(agg, cnt, h, w2, b2, skw, skb, g, beta, hw1, hb1, hw2, hb2):
    pd = hw2.shape[1]
    wspec = pl.BlockSpec((_EMB, _EMB), lambda i: (0, 0))
    vspec = pl.BlockSpec((1, _EMB), lambda i: (0, 0))
    return pl.pallas_call(
        _fin_body,
        grid=(_N // _NB,),
        in_specs=[
            pl.BlockSpec((2, _NB, _HALF), lambda i: (0, i, 0)),
            pl.BlockSpec((2, _NB, _HALF), lambda i: (0, i, 0)),
            pl.BlockSpec((_NB, _EMB), lambda i: (i, 0)),
            wspec, vspec, wspec, vspec, vspec, vspec,
            wspec, vspec,
            pl.BlockSpec((_EMB, pd), lambda i: (0, 0)),
            pl.BlockSpec((1, pd), lambda i: (0, 0)),
        ],
        out_specs=[pl.BlockSpec((_NB, pd), lambda i: (i, 0))],
        out_shape=[jax.ShapeDtypeStruct((_N, pd), jnp.float32)],
    )(agg, cnt, h, w2, b2, skw, skb, g, beta, hw1, hb1, hw2, hb2)[0]


def kernel(node_features, edge_index, edge_features, n,
           enc_W1, enc_b1, enc_W2, enc_b2,
           conv_W1, conv_b1, conv_W2, conv_b2,
           skip_W, skip_b, ln_g, ln_b,
           head_W1, head_b1, head_W2, head_b2):
    rows = edge_index[0].astype(jnp.int32)
    cols = edge_index[1].astype(jnp.int32)
    ef0 = edge_features[:, 0]
    ef1 = edge_features[:, 1]

    h, a_tab, b_tab = _enc_call(
        node_features, enc_W1, enc_b1.reshape(1, -1), enc_W2,
        enc_b2.reshape(1, -1), conv_W1[0, :_EMB], conv_W1[0, _EMB:2 * _EMB],
        conv_b1[0].reshape(1, -1))

    wv = jnp.stack([conv_W1[:, 2 * _EMB:, :_HALF],
                    conv_W1[:, 2 * _EMB:, _HALF:]], axis=1)

    cnt = _sc_count_pass(rows).reshape(2, _N, _HALF)

    out = None
    for i in range(_NLAYERS):
        agg = _sc_edge_pass(
            a_tab.reshape(2 * _N, _HALF),
            b_tab.reshape(2 * _N, _HALF),
            ef0, ef1, wv[i],
            rows, cols).reshape(2, _N, _HALF)
        if i < _NLAYERS - 1:
            h, a_tab, b_tab = _upd_call(
                agg, cnt, h, conv_W2[i], conv_b2[i].reshape(1, -1),
                skip_W[i], skip_b[i].reshape(1, -1),
                ln_g[i].reshape(1, -1), ln_b[i].reshape(1, -1),
                conv_W1[i + 1, :_EMB], conv_W1[i + 1, _EMB:2 * _EMB],
                conv_b1[i + 1].reshape(1, -1))
        else:
            out = _fin_call(
                agg, cnt, h, conv_W2[i], conv_b2[i].reshape(1, -1),
                skip_W[i], skip_b[i].reshape(1, -1),
                ln_g[i].reshape(1, -1), ln_b[i].reshape(1, -1),
                head_W1, head_b1.reshape(1, -1),
                head_W2, head_b2.reshape(1, -1))
    return out


# speedup vs baseline: 1.4224x; 1.4224x over previous
import functools

import jax
import jax.numpy as jnp
from jax import lax
from jax.experimental import pallas as pl
from jax.experimental.pallas import tpu as pltpu
from jax.experimental.pallas import tpu_sc as plsc

_N = 100000
_NE = 1600000
_EMB = 32
_HALF = 16
_NLAYERS = 3
_TILES = 16

_KE = 400
_ECHUNKS = _NE // _TILES // _KE
_KC = 1000
_CCHUNKS = _NE // 2 // _TILES // _KC
_ZCH = _N // _KE
_ZITERS = -(-_ZCH // _TILES)

_NB = 4000
_EBLK = 2000

_sc_mesh = plsc.VectorSubcoreMesh(core_axis_name="c", subcore_axis_name="s")
_sc_params = pltpu.CompilerParams(use_tc_tiling_on_sc=False)


@functools.partial(
    pl.kernel,
    out_type=jax.ShapeDtypeStruct((2 * _N, _HALF), jnp.float32),
    mesh=_sc_mesh,
    compiler_params=_sc_params,
    scratch_types=[
        pltpu.VMEM((2, _KE), jnp.int32),
        pltpu.VMEM((_KE,), jnp.int32),
        pltpu.VMEM((_KE,), jnp.int32),
        pltpu.VMEM((_KE, _HALF), jnp.float32),
        pltpu.VMEM((_KE, _HALF), jnp.float32),
        pltpu.VMEM((_KE,), jnp.float32),
        pltpu.VMEM((_KE,), jnp.float32),
        pltpu.VMEM((_KE, _HALF), jnp.float32),
        pltpu.VMEM((2, _HALF), jnp.float32),
        pltpu.SemaphoreType.DMA,
        pltpu.SemaphoreType.DMA,
        pltpu.SemaphoreType.DMA,
        pltpu.SemaphoreType.DMA,
        pltpu.VMEM_SHARED((_N, _HALF), jnp.float32),
    ],
)
def _sc_edge_pass(a_tab, b_tab, ef0_hbm, ef1_hbm, wv_hbm, rows_hbm, cols_hbm,
                  out_hbm,
                  idx_r, idx_ro, idx_co, abuf, bbuf, e0buf, e1buf, mbuf, wbuf,
                  sem_a, sem_b, sem_e, sem_s, acc):
    c = lax.axis_index("c")
    t = lax.axis_index("s")
    pltpu.sync_copy(wv_hbm.at[c], wbuf)
    w0 = wbuf[0]
    w1 = wbuf[1]

    def _zrow(j, carry):
        mbuf[j] = jnp.zeros((_HALF,), jnp.float32)
        return carry
    lax.fori_loop(0, _KE, _zrow, 0)

    def _zchunk(m_i, carry):
        m = m_i * _TILES + t

        @pl.when(m < _ZCH)
        def _():
            pltpu.sync_copy(mbuf, acc.at[pl.ds(m * _KE, _KE)])
        return carry
    lax.fori_loop(0, _ZITERS, _zchunk, 0)
    plsc.subcore_barrier()

    ebase = t * (_NE // _TILES)
    coff = c * _N

    def _load_idx_and_start_gathers(k, p):
        off = ebase + k * _KE
        cp_r = pltpu.async_copy(rows_hbm.at[pl.ds(off, _KE)], idx_r.at[p], sem_a)
        cp_c = pltpu.async_copy(cols_hbm.at[pl.ds(off, _KE)], idx_co, sem_b)
        cp_r.wait()
        cp_c.wait()
        for j in range(_KE // 16):
            s = pl.ds(j * 16, 16)
            idx_ro[s] = idx_r[p, s] + coff
            idx_co[s] = idx_co[s] + coff
        pltpu.async_copy(a_tab.at[idx_ro], abuf, sem_a)
        pltpu.async_copy(b_tab.at[idx_co], bbuf, sem_b)
        pltpu.async_copy(ef0_hbm.at[pl.ds(off, _KE)], e0buf, sem_e)
        pltpu.async_copy(ef1_hbm.at[pl.ds(off, _KE)], e1buf, sem_e)

    _load_idx_and_start_gathers(0, 0)

    def _chunk(k, carry):
        p = lax.rem(k, 2)
        pltpu.make_async_copy(a_tab.at[idx_ro], abuf, sem_a).wait()
        pltpu.make_async_copy(b_tab.at[idx_co], bbuf, sem_b).wait()
        pltpu.make_async_copy(ef0_hbm.at[pl.ds(0, _KE)], e0buf, sem_e).wait()
        pltpu.make_async_copy(ef1_hbm.at[pl.ds(0, _KE)], e1buf, sem_e).wait()

        @pl.when(k > 0)
        def _():
            pltpu.make_async_copy(mbuf, acc.at[idx_r.at[1 - p]], sem_s).wait()

        def _comp(g, cc):
            es0 = e0buf[pl.ds(g * 16, 16)]
            es1 = e1buf[pl.ds(g * 16, 16)]
            for u in range(16):
                j = g * 16 + u
                ev = es0[u] * w0 + es1[u] * w1
                mbuf[j] = jnp.maximum(abuf[j] + bbuf[j] + ev, 0.0)
            return cc
        lax.fori_loop(0, _KE // 16, _comp, 0)

        pltpu.async_copy(mbuf, acc.at[idx_r.at[p]], sem_s, add=True)

        @pl.when(k < _ECHUNKS - 1)
        def _():
            _load_idx_and_start_gathers(k + 1, 1 - p)
        return carry
    lax.fori_loop(0, _ECHUNKS, _chunk, 0)
    pltpu.make_async_copy(
        mbuf, acc.at[idx_r.at[(_ECHUNKS - 1) % 2]], sem_s).wait()
    plsc.subcore_barrier()

    def _wchunk(m_i, carry):
        m = m_i * _TILES + t

        @pl.when(m < _ZCH)
        def _():
            pltpu.sync_copy(acc.at[pl.ds(m * _KE, _KE)],
                            out_hbm.at[pl.ds(c * _N + m * _KE, _KE)])
        return carry
    lax.fori_loop(0, _ZITERS, _wchunk, 0)


@functools.partial(
    pl.kernel,
    out_type=jax.ShapeDtypeStruct((2 * _N, _HALF), jnp.float32),
    mesh=_sc_mesh,
    compiler_params=_sc_params,
    scratch_types=[
        pltpu.VMEM((_KC,), jnp.int32),
        pltpu.VMEM((_KC, _HALF), jnp.float32),
        pltpu.SemaphoreType.DMA,
        pltpu.VMEM_SHARED((_N, _HALF), jnp.float32),
    ],
)
def _sc_count_pass(rows_hbm, out_hbm, idx_r, vbuf, sem, acc):
    c = lax.axis_index("c")
    t = lax.axis_index("s")

    def _zrow(j, carry):
        vbuf[j] = jnp.zeros((_HALF,), jnp.float32)
        return carry
    lax.fori_loop(0, _KC, _zrow, 0)

    def _zchunk(m_i, carry):
        m = m_i * _TILES + t

        @pl.when(m < _N // _KC)
        def _():
            pltpu.sync_copy(vbuf, acc.at[pl.ds(m * _KC, _KC)])
        return carry
    lax.fori_loop(0, -(-(_N // _KC) // _TILES), _zchunk, 0)
    plsc.subcore_barrier()

    def _orow(j, carry):
        vbuf[j] = jnp.full((_HALF,), 1.0, jnp.float32)
        return carry
    lax.fori_loop(0, _KC, _orow, 0)

    ebase = c * (_NE // 2) + t * (_NE // 2 // _TILES)

    def _chunk(k, carry):
        off = ebase + k * _KC
        pltpu.sync_copy(rows_hbm.at[pl.ds(off, _KC)], idx_r)
        pltpu.sync_copy(vbuf, acc.at[idx_r], add=True)
        return carry
    lax.fori_loop(0, _CCHUNKS, _chunk, 0)
    plsc.subcore_barrier()

    def _wchunk(m_i, carry):
        m = m_i * _TILES + t

        @pl.when(m < _N // _KC)
        def _():
            pltpu.sync_copy(acc.at[pl.ds(m * _KC, _KC)],
                            out_hbm.at[pl.ds(c * _N + m * _KC, _KC)])
        return carry
    lax.fori_loop(0, -(-(_N // _KC) // _TILES), _wchunk, 0)


def _enc_body(nf_ref, w1, b1, w2, b2, wr, wc, cb1, h_ref, a_ref, b_ref):
    x = nf_ref[...]
    tmp = jnp.maximum(
        jnp.dot(x, w1[...], preferred_element_type=jnp.float32) + b1[...], 0.0)
    h = jnp.dot(tmp, w2[...], preferred_element_type=jnp.float32) + b2[...]
    h_ref[...] = h
    p = jnp.dot(h, wr[...], preferred_element_type=jnp.float32) + cb1[...]
    q = jnp.dot(h, wc[...], preferred_element_type=jnp.float32)
    a_ref[0] = p[:, :_HALF]
    a_ref[1] = p[:, _HALF:]
    b_ref[0] = q[:, :_HALF]
    b_ref[1] = q[:, _HALF:]


def _enc_call(nf, w1, b1, w2, b2, wr, wc, cb1):
    hid = w1.shape[1]
    return pl.pallas_call(
        _enc_body,
        grid=(_N // _NB,),
        in_specs=[
            pl.BlockSpec((_NB, nf.shape[1]), lambda i: (i, 0)),
            pl.BlockSpec(w1.shape, lambda i: (0, 0)),
            pl.BlockSpec((1, hid), lambda i: (0, 0)),
            pl.BlockSpec(w2.shape, lambda i: (0, 0)),
            pl.BlockSpec((1, _EMB), lambda i: (0, 0)),
            pl.BlockSpec((_EMB, _EMB), lambda i: (0, 0)),
            pl.BlockSpec((_EMB, _EMB), lambda i: (0, 0)),
            pl.BlockSpec((1, _EMB), lambda i: (0, 0)),
        ],
        out_specs=[
            pl.BlockSpec((_NB, _EMB), lambda i: (i, 0)),
            pl.BlockSpec((2, _NB, _HALF), lambda i: (0, i, 0)),
            pl.BlockSpec((2, _NB, _HALF), lambda i: (0, i, 0)),
        ],
        out_shape=[
            jax.ShapeDtypeStruct((_N, _EMB), jnp.float32),
            jax.ShapeDtypeStruct((2, _N, _HALF), jnp.float32),
            jax.ShapeDtypeStruct((2, _N, _HALF), jnp.float32),
        ],
    )(nf, w1, b1, w2, b2, wr, wc, cb1)


def _node_update(agg_ref, cnt_ref, h_ref, w2, b2, skw, skb, g, beta):
    agg = jnp.concatenate([agg_ref[0], agg_ref[1]], axis=-1)
    cv = cnt_ref[0, :, 0:1] + cnt_ref[1, :, 0:1]
    h = h_ref[...]
    z = (jnp.dot(agg, w2[...], preferred_element_type=jnp.float32)
         + cv * b2[...]
         + jnp.dot(h, skw[...], preferred_element_type=jnp.float32)
         + skb[...])
    mu = jnp.mean(z, axis=-1, keepdims=True)
    var = jnp.mean(jnp.square(z - mu), axis=-1, keepdims=True)
    zn = (z - mu) * lax.rsqrt(var + 1e-5) * g[...] + beta[...]
    return jnp.maximum(zn, 0.0)


def _upd_body(agg_ref, cnt_ref, h_ref, w2, b2, skw, skb, g, beta, wr, wc, cb1,
              hn_ref, a_ref, b_ref):
    hn = _node_update(agg_ref, cnt_ref, h_ref, w2, b2, skw, skb, g, beta)
    hn_ref[...] = hn
    p = jnp.dot(hn, wr[...], preferred_element_type=jnp.float32) + cb1[...]
    q = jnp.dot(hn, wc[...], preferred_element_type=jnp.float32)
    a_ref[0] = p[:, :_HALF]
    a_ref[1] = p[:, _HALF:]
    b_ref[0] = q[:, :_HALF]
    b_ref[1] = q[:, _HALF:]


def _upd_call(agg, cnt, h, w2, b2, skw, skb, g, beta, wr, wc, cb1):
    wspec = pl.BlockSpec((_EMB, _EMB), lambda i: (0, 0))
    vspec = pl.BlockSpec((1, _EMB), lambda i: (0, 0))
    return pl.pallas_call(
        _upd_body,
        grid=(_N // _NB,),
        in_specs=[
            pl.BlockSpec((2, _NB, _HALF), lambda i: (0, i, 0)),
            pl.BlockSpec((2, _NB, _HALF), lambda i: (0, i, 0)),
            pl.BlockSpec((_NB, _EMB), lambda i: (i, 0)),
            wspec, vspec, wspec, vspec, vspec, vspec, wspec, wspec, vspec,
        ],
        out_specs=[
            pl.BlockSpec((_NB, _EMB), lambda i: (i, 0)),
            pl.BlockSpec((2, _NB, _HALF), lambda i: (0, i, 0)),
            pl.BlockSpec((2, _NB, _HALF), lambda i: (0, i, 0)),
        ],
        out_shape=[
            jax.ShapeDtypeStruct((_N, _EMB), jnp.float32),
            jax.ShapeDtypeStruct((2, _N, _HALF), jnp.float32),
            jax.ShapeDtypeStruct((2, _N, _HALF), jnp.float32),
        ],
    )(agg, cnt, h, w2, b2, skw, skb, g, beta, wr, wc, cb1)


def _fin_body(agg_ref, cnt_ref, h_ref, w2, b2, skw, skb, g, beta,
              hw1, hb1, hw2, hb2, out_ref):
    hn = _node_update(agg_ref, cnt_ref, h_ref, w2, b2, skw, skb, g, beta)
    tt = jnp.maximum(
        jnp.dot(hn, hw1[...], preferred_element_type=jnp.float32) + hb1[...],
        0.0)
    out_ref[...] = jnp.dot(tt, hw2[...], preferred_element_type=jnp.float32) + hb2[...]


def _fin_call(agg, cnt, h, w2, b2, skw, skb, g, beta, hw1, hb1, hw2, hb2):
    pd = hw2.shape[1]
    wspec = pl.BlockSpec((_EMB, _EMB), lambda i: (0, 0))
    vspec = pl.BlockSpec((1, _EMB), lambda i: (0, 0))
    return pl.pallas_call(
        _fin_body,
        grid=(_N // _NB,),
        in_specs=[
            pl.BlockSpec((2, _NB, _HALF), lambda i: (0, i, 0)),
            pl.BlockSpec((2, _NB, _HALF), lambda i: (0, i, 0)),
            pl.BlockSpec((_NB, _EMB), lambda i: (i, 0)),
            wspec, vspec, wspec, vspec, vspec, vspec,
            wspec, vspec,
            pl.BlockSpec((_EMB, pd), lambda i: (0, 0)),
            pl.BlockSpec((1, pd), lambda i: (0, 0)),
        ],
        out_specs=[pl.BlockSpec((_NB, pd), lambda i: (i, 0))],
        out_shape=[jax.ShapeDtypeStruct((_N, pd), jnp.float32)],
    )(agg, cnt, h, w2, b2, skw, skb, g, beta, hw1, hb1, hw2, hb2)[0]


def kernel(node_features, edge_index, edge_features, n,
           enc_W1, enc_b1, enc_W2, enc_b2,
           conv_W1, conv_b1, conv_W2, conv_b2,
           skip_W, skip_b, ln_g, ln_b,
           head_W1, head_b1, head_W2, head_b2):
    rows = edge_index[0].astype(jnp.int32)
    cols = edge_index[1].astype(jnp.int32)
    ef0 = edge_features[:, 0]
    ef1 = edge_features[:, 1]

    h, a_tab, b_tab = _enc_call(
        node_features, enc_W1, enc_b1.reshape(1, -1), enc_W2,
        enc_b2.reshape(1, -1), conv_W1[0, :_EMB], conv_W1[0, _EMB:2 * _EMB],
        conv_b1[0].reshape(1, -1))

    wv = jnp.stack([conv_W1[:, 2 * _EMB:, :_HALF],
                    conv_W1[:, 2 * _EMB:, _HALF:]], axis=1)

    cnt = _sc_count_pass(rows).reshape(2, _N, _HALF)

    out = None
    for i in range(_NLAYERS):
        agg = _sc_edge_pass(
            a_tab.reshape(2 * _N, _HALF),
            b_tab.reshape(2 * _N, _HALF),
            ef0, ef1, wv[i],
            rows, cols).reshape(2, _N, _HALF)
        if i < _NLAYERS - 1:
            h, a_tab, b_tab = _upd_call(
                agg, cnt, h, conv_W2[i], conv_b2[i].reshape(1, -1),
                skip_W[i], skip_b[i].reshape(1, -1),
                ln_g[i].reshape(1, -1), ln_b[i].reshape(1, -1),
                conv_W1[i + 1, :_EMB], conv_W1[i + 1, _EMB:2 * _EMB],
                conv_b1[i + 1].reshape(1, -1))
        else:
            out = _fin_call(
                agg, cnt, h, conv_W2[i], conv_b2[i].reshape(1, -1),
                skip_W[i], skip_b[i].reshape(1, -1),
                ln_g[i].reshape(1, -1), ln_b[i].reshape(1, -1),
                head_W1, head_b1.reshape(1, -1),
                head_W2, head_b2.reshape(1, -1))
    return out
